# baseline (speedup 1.0000x reference)
import functools

import jax
import jax.numpy as jnp
from jax import lax
from jax.experimental import pallas as pl
from jax.experimental.pallas import tpu as pltpu
from jax.experimental.pallas import tpu_sc as plsc

NE = 8192
D = 256
NT = 16384
TM = 256
TN = 2048
GRID = NT // TM
NCH = NE // TN
STRIPS = (0, 2736, 5472, NE)
BIG = 2 ** 30


def _bf16_rne(x):
    b = lax.bitcast_convert_type(x, jnp.uint32)
    r = b + jnp.uint32(0x7FFF) + ((b >> 16) & jnp.uint32(1))
    return lax.bitcast_convert_type(r & jnp.uint32(0xFFFF0000), jnp.float32)


def _dist_argmin_body(z_ref, wt_ref, zsq_ref, wsq_ref, idx_ref, loss_ref):
    i = pl.program_id(0)
    z = z_ref[...]
    zsq = zsq_ref[...]
    z_bf = z.astype(jnp.bfloat16)

    ns = len(STRIPS) - 1
    smin = [jnp.full((TM, 1), jnp.inf, jnp.float32) for _ in range(ns)]
    sidx = [jnp.zeros((TM, 1), jnp.int32) for _ in range(ns)]

    for c in range(NCH):
        c0, c1 = c * TN, (c + 1) * TN
        wt_c = wt_ref[:, c0:c1].astype(jnp.bfloat16)
        s = lax.dot_general(z_bf, wt_c, (((1,), (0,)), ((), ())),
                            preferred_element_type=jnp.float32)
        d = (zsq - 2.0 * s) + wsq_ref[:, c0:c1]
        col = lax.broadcasted_iota(jnp.int32, (TM, TN), 1) + c0
        for st in range(ns):
            lo, hi = max(STRIPS[st], c0), min(STRIPS[st + 1], c1)
            if lo >= hi:
                continue
            if lo == c0 and hi == c1:
                dm = d
            else:
                inside = (col >= lo) & (col < hi)
                dm = jnp.where(inside, d, jnp.inf)
            m = jnp.min(dm, axis=1, keepdims=True)
            idx_c = jnp.min(jnp.where(dm == m, col, BIG),
                            axis=1, keepdims=True)
            better = m < smin[st]
            smin[st] = jnp.where(better, m, smin[st])
            sidx[st] = jnp.where(better, idx_c, sidx[st])

    accv = _bf16_rne(smin[0])
    acci = sidx[0]
    for st in range(1, ns):
        take = smin[st] < accv
        accv = jnp.where(take, _bf16_rne(smin[st]), accv)
        acci = jnp.where(take, sidx[st], acci)

    idx_ref[...] = acci

    @pl.when(i == 0)
    def _():
        loss_ref[0, 0] = 0.0

    loss_ref[0, 0] += jnp.sum(jnp.minimum(jnp.minimum(smin[0], smin[1]),
                                          smin[2]))


def _argmin_tc(ze_flat, wt, zsq, wsq):
    return pl.pallas_call(
        _dist_argmin_body,
        grid=(GRID,),
        in_specs=[
            pl.BlockSpec((TM, D), lambda i: (i, 0)),
            pl.BlockSpec((D, NE), lambda i: (0, 0)),
            pl.BlockSpec((TM, 1), lambda i: (i, 0)),
            pl.BlockSpec((1, NE), lambda i: (0, 0)),
        ],
        out_specs=[
            pl.BlockSpec((TM, 1), lambda i: (i, 0)),
            pl.BlockSpec((1, 1), lambda i: (0, 0), memory_space=pltpu.SMEM),
        ],
        out_shape=[
            jax.ShapeDtypeStruct((NT, 1), jnp.int32),
            jax.ShapeDtypeStruct((1, 1), jnp.float32),
        ],
    )(ze_flat, wt, zsq, wsq)


_GCH = 128


def _sc_gather(table, idx):
    info = plsc.get_sparse_core_info()
    nw = info.num_cores * info.num_subcores
    bpw = NT // nw
    mesh = plsc.VectorSubcoreMesh(core_axis_name="c", subcore_axis_name="s")

    @functools.partial(
        pl.kernel, mesh=mesh,
        out_type=jax.ShapeDtypeStruct((NT, D), jnp.float32),
        scratch_types=[
            pltpu.VMEM((_GCH,), jnp.int32),
            pltpu.VMEM((_GCH, D), jnp.float32),
            pltpu.SemaphoreType.DMA,
        ],
    )
    def gk(table_hbm, idx_hbm, out_hbm, idx_v, rows_v, sem):
        wid = lax.axis_index("s") * info.num_cores + lax.axis_index("c")
        base = wid * bpw
        for c in range(bpw // _GCH):
            off = base + c * _GCH
            pltpu.sync_copy(idx_hbm.at[pl.ds(off, _GCH)], idx_v)
            pltpu.async_copy(table_hbm.at[idx_v], rows_v, sem).wait()
            pltpu.sync_copy(rows_v, out_hbm.at[pl.ds(off, _GCH)])

    return gk(table, idx)


def kernel(ze, embedW):
    B, T, _ = ze.shape
    ze_flat = ze.reshape(NT, D)
    zsq = lax.optimization_barrier((ze ** 2).sum(axis=2)).reshape(NT, 1)
    wsq = lax.optimization_barrier((embedW ** 2).sum(axis=1)).reshape(1, NE)
    wt = embedW.T
    idx2, loss_sum = _argmin_tc(ze_flat, wt, zsq, wsq)
    idx_flat = idx2.reshape(NT)
    zq_flat = _sc_gather(embedW, idx_flat)
    zq_out = zq_flat.reshape(B, T, D)
    dist_loss = loss_sum[0, 0] / jnp.float32(NT * D)
    zq_idx = idx_flat.reshape(B, T)
    return (zq_out, dist_loss, zq_idx)


# speedup vs baseline: 1.0328x; 1.0328x over previous
import functools

import jax
import jax.numpy as jnp
from jax import lax
from jax.experimental import pallas as pl
from jax.experimental.pallas import tpu as pltpu
from jax.experimental.pallas import tpu_sc as plsc

NE = 8192
D = 256
NT = 16384
TM = 256
TN = 2048
GRID = NT // TM
NCH = NE // TN
STRIPS = (0, 2736, 5472, NE)
BIG = 2 ** 30


def _bf16_rne(x):
    b = lax.bitcast_convert_type(x, jnp.uint32)
    r = b + jnp.uint32(0x7FFF) + ((b >> 16) & jnp.uint32(1))
    return lax.bitcast_convert_type(r & jnp.uint32(0xFFFF0000), jnp.float32)


def _dist_argmin_body(z_ref, wt_ref, zsq_ref, wsq_ref, idx_ref, loss_ref):
    i = pl.program_id(0)
    z = z_ref[...]
    zsq = zsq_ref[...]
    z_bf = z.astype(jnp.bfloat16)

    ns = len(STRIPS) - 1
    smin = [jnp.full((TM, 1), jnp.inf, jnp.float32) for _ in range(ns)]
    sidx = [jnp.zeros((TM, 1), jnp.float32) for _ in range(ns)]

    col = lax.broadcasted_iota(jnp.int32, (TM, TN), 1).astype(jnp.float32)
    for c in range(NCH):
        c0, c1 = c * TN, (c + 1) * TN
        wt_c = wt_ref[:, c0:c1]
        s = lax.dot_general(z_bf, wt_c, (((1,), (0,)), ((), ())),
                            preferred_element_type=jnp.float32)
        d = (zsq - 2.0 * s) + wsq_ref[:, c0:c1]
        for st in range(ns):
            lo, hi = max(STRIPS[st], c0), min(STRIPS[st + 1], c1)
            if lo >= hi:
                continue
            if lo == c0 and hi == c1:
                dm = d
            else:
                inside = (col >= float(lo - c0)) & (col < float(hi - c0))
                dm = jnp.where(inside, d, jnp.inf)
            m = jnp.min(dm, axis=1, keepdims=True)
            idx_c = jnp.min(jnp.where(dm == m, col, jnp.inf),
                            axis=1, keepdims=True) + float(c0)
            better = m < smin[st]
            smin[st] = jnp.where(better, m, smin[st])
            sidx[st] = jnp.where(better, idx_c, sidx[st])

    accv = _bf16_rne(smin[0])
    acci = sidx[0]
    for st in range(1, ns):
        take = smin[st] < accv
        accv = jnp.where(take, _bf16_rne(smin[st]), accv)
        acci = jnp.where(take, sidx[st], acci)

    idx_ref[...] = acci.astype(jnp.int32)

    @pl.when(i == 0)
    def _():
        loss_ref[0, 0] = 0.0

    loss_ref[0, 0] += jnp.sum(jnp.minimum(jnp.minimum(smin[0], smin[1]),
                                          smin[2]))


def _argmin_tc(ze_flat, wt, zsq, wsq):
    return pl.pallas_call(
        _dist_argmin_body,
        grid=(GRID,),
        in_specs=[
            pl.BlockSpec((TM, D), lambda i: (i, 0)),
            pl.BlockSpec((D, NE), lambda i: (0, 0)),
            pl.BlockSpec((TM, 1), lambda i: (i, 0)),
            pl.BlockSpec((1, NE), lambda i: (0, 0)),
        ],
        out_specs=[
            pl.BlockSpec((TM, 1), lambda i: (i, 0)),
            pl.BlockSpec((1, 1), lambda i: (0, 0), memory_space=pltpu.SMEM),
        ],
        out_shape=[
            jax.ShapeDtypeStruct((NT, 1), jnp.int32),
            jax.ShapeDtypeStruct((1, 1), jnp.float32),
        ],
    )(ze_flat, wt, zsq, wsq)


_GCH = 128


def _sc_gather(table, idx):
    info = plsc.get_sparse_core_info()
    nw = info.num_cores * info.num_subcores
    bpw = NT // nw
    mesh = plsc.VectorSubcoreMesh(core_axis_name="c", subcore_axis_name="s")

    @functools.partial(
        pl.kernel, mesh=mesh,
        out_type=jax.ShapeDtypeStruct((NT, D), jnp.float32),
        scratch_types=[
            pltpu.VMEM((_GCH,), jnp.int32),
            pltpu.VMEM((_GCH, D), jnp.float32),
            pltpu.SemaphoreType.DMA,
        ],
    )
    def gk(table_hbm, idx_hbm, out_hbm, idx_v, rows_v, sem):
        wid = lax.axis_index("s") * info.num_cores + lax.axis_index("c")
        base = wid * bpw
        for c in range(bpw // _GCH):
            off = base + c * _GCH
            pltpu.sync_copy(idx_hbm.at[pl.ds(off, _GCH)], idx_v)
            pltpu.async_copy(table_hbm.at[idx_v], rows_v, sem).wait()
            pltpu.sync_copy(rows_v, out_hbm.at[pl.ds(off, _GCH)])

    return gk(table, idx)


def kernel(ze, embedW):
    B, T, _ = ze.shape
    ze_flat = ze.reshape(NT, D)
    zsq = lax.optimization_barrier((ze ** 2).sum(axis=2)).reshape(NT, 1)
    wsq = lax.optimization_barrier((embedW ** 2).sum(axis=1)).reshape(1, NE)
    wt = embedW.T.astype(jnp.bfloat16)
    idx2, loss_sum = _argmin_tc(ze_flat, wt, zsq, wsq)
    idx_flat = idx2.reshape(NT)
    zq_flat = _sc_gather(embedW, idx_flat)
    zq_out = zq_flat.reshape(B, T, D)
    dist_loss = loss_sum[0, 0] / jnp.float32(NT * D)
    zq_idx = idx_flat.reshape(B, T)
    return (zq_out, dist_loss, zq_idx)
